# baseline (speedup 1.0000x reference)
import functools

import jax
import jax.numpy as jnp
from jax import lax
from jax.experimental import pallas as pl
from jax.experimental.pallas import tpu as pltpu
from jax.experimental.pallas import tpu_sc as plsc

B, C, N, K, O = 4, 64, 4096, 20, 64
KPAD = 32
T = 256
T2 = 128
T3 = 512
NW = 32
TOT = B * K * N
ROWS_PER_W = TOT // NW
CH = 128
NCH = ROWS_PER_W // CH


def _knn_body(xf_ref, xt_ref, w_ref, idx_ref, y1_ref, y2_ref, s_ref):
    b = pl.program_id(0)
    xb = xf_ref[0]
    xs = xt_ref[0]
    w = w_ref[...]
    inner = lax.dot_general(xs, xb, (((0,), (0,)), ((), ())),
                            preferred_element_type=jnp.float32)
    nf = jnp.sum(xb * xb, axis=0, keepdims=True)
    nt = jnp.sum(xs * xs, axis=0)[:, None]
    s_ref[...] = (2.0 * inner - nt) - nf

    w1pad = jnp.concatenate([w[:, :C], jnp.zeros((O, C), jnp.float32)], axis=0)
    y1_ref[0] = lax.dot_general(xs, w1pad, (((0,), (1,)), ((), ())),
                                preferred_element_type=jnp.float32)
    y2_ref[0] = lax.dot_general(xs, w[:, C:], (((0,), (1,)), ((), ())),
                                preferred_element_type=jnp.float32)

    col = lax.broadcasted_iota(jnp.int32, (T, N), 1)
    base = b * N

    def body(k, aprev):
        s = jnp.where(col == aprev[:, None], -jnp.inf, s_ref[...])
        s_ref[...] = s
        mx = jnp.max(s, axis=1)
        a = jnp.min(jnp.where(s == mx[:, None], col, N),
                    axis=1).astype(jnp.int32)
        idx_ref[0, pl.ds(k, 1), :] = (a + base)[None, :]
        return a

    lax.fori_loop(0, K, body, jnp.full((T,), -1, jnp.int32))


_knn_call = pl.pallas_call(
    _knn_body,
    grid=(B, N // T),
    in_specs=[
        pl.BlockSpec((1, C, N), lambda b, j: (b, 0, 0)),
        pl.BlockSpec((1, C, T), lambda b, j: (b, 0, j)),
        pl.BlockSpec((O, 2 * C), lambda b, j: (0, 0)),
    ],
    out_specs=[
        pl.BlockSpec((1, KPAD, T), lambda b, j: (b, 0, j)),
        pl.BlockSpec((1, T, 2 * O), lambda b, j: (b, j, 0)),
        pl.BlockSpec((1, T, O), lambda b, j: (b, j, 0)),
    ],
    out_shape=[
        jax.ShapeDtypeStruct((B, KPAD, N), jnp.int32),
        jax.ShapeDtypeStruct((B, N, 2 * O), jnp.float32),
        jax.ShapeDtypeStruct((B, N, O), jnp.float32),
    ],
    scratch_shapes=[pltpu.VMEM((T, N), jnp.float32)],
)


@functools.cache
def _make_sc_gather():
    return functools.partial(
        pl.kernel,
        mesh=plsc.VectorSubcoreMesh(core_axis_name="c", subcore_axis_name="s"),
        out_type=jax.ShapeDtypeStruct((TOT, 2 * O), jnp.float32),
        scratch_types=[
            pltpu.VMEM((NCH, CH), jnp.int32),
            pltpu.VMEM((CH, 2 * O), jnp.float32),
            pltpu.VMEM((CH, 2 * O), jnp.float32),
            pltpu.SemaphoreType.DMA,
            pltpu.SemaphoreType.DMA,
        ],
    )(_sc_gather_body)


def _sc_gather_body(table_hbm, idx_hbm, out_hbm, idx_v, buf0, buf1, sem0, sem1):
    wid = lax.axis_index("s") * 2 + lax.axis_index("c")
    pltpu.sync_copy(idx_hbm.at[pl.ds(wid * NCH, NCH)], idx_v)
    base = wid * ROWS_PER_W

    def start(i, buf, sem):
        return pltpu.async_copy(table_hbm.at[idx_v.at[i]], buf, sem)

    def drain(i, buf, cp):
        cp.wait()
        off = pl.multiple_of(base + i * CH, CH)
        pltpu.sync_copy(buf, out_hbm.at[pl.ds(off, CH)])

    def body(i, _):
        cp0 = start(2 * i, buf0, sem0)
        cp1 = start(2 * i + 1, buf1, sem1)
        drain(2 * i, buf0, cp0)
        drain(2 * i + 1, buf1, cp1)
        return 0

    lax.fori_loop(0, NCH // 2, body, 0)


def _pool_body(g_ref, y2_ref, m_ref, st_ref):
    y2 = y2_ref[0]
    m = jnp.full((T2, O), -jnp.inf, jnp.float32)
    zs = jnp.zeros((T2, O), jnp.float32)
    zq = jnp.zeros((T2, O), jnp.float32)
    for k in range(K):
        z = g_ref[0, k, :, :O] + y2
        m = jnp.maximum(m, z)
        zs = zs + z
        zq = zq + z * z
    m_ref[0] = m
    s = jnp.sum(zs, axis=0)
    q = jnp.sum(zq, axis=0)
    zero = jnp.zeros((O,), jnp.float32)
    blk = jnp.concatenate([
        jnp.concatenate([s, zero])[None, :],
        jnp.concatenate([q, zero])[None, :],
        jnp.zeros((6, 128), jnp.float32),
    ], axis=0)
    first = (pl.program_id(0) == 0) & (pl.program_id(1) == 0)

    @pl.when(first)
    def _():
        st_ref[...] = jnp.zeros((8, 128), jnp.float32)

    st_ref[...] += blk


_pool_call = pl.pallas_call(
    _pool_body,
    grid=(B, N // T2),
    in_specs=[
        pl.BlockSpec((1, K, T2, 2 * O), lambda b, j: (b, 0, j, 0)),
        pl.BlockSpec((1, T2, O), lambda b, j: (b, j, 0)),
    ],
    out_specs=[
        pl.BlockSpec((1, T2, O), lambda b, j: (b, j, 0)),
        pl.BlockSpec((8, 128), lambda b, j: (0, 0)),
    ],
    out_shape=[
        jax.ShapeDtypeStruct((B, N, O), jnp.float32),
        jax.ShapeDtypeStruct((8, 128), jnp.float32),
    ],
)


def _norm_body(m_ref, st_ref, aux_ref, o_ref):
    m = m_ref[0]
    s = st_ref[0, :O]
    q = st_ref[1, :O]
    gamma = aux_ref[0, :O]
    beta = aux_ref[1, :O]
    cnt = jnp.float32(B * N * K)
    mean = s / cnt
    var = q / cnt - mean * mean
    inv = lax.rsqrt(var + 1e-5)
    scale = inv * gamma
    shift = beta - mean * scale
    o = m * scale[None, :] + shift[None, :]
    o_ref[0] = jnp.where(o > 0, o, 0.2 * o)


_norm_call = pl.pallas_call(
    _norm_body,
    grid=(B, N // T3),
    in_specs=[
        pl.BlockSpec((1, T3, O), lambda b, j: (b, j, 0)),
        pl.BlockSpec((8, 128), lambda b, j: (0, 0)),
        pl.BlockSpec((8, 128), lambda b, j: (0, 0)),
    ],
    out_specs=pl.BlockSpec((1, T3, O), lambda b, j: (b, j, 0)),
    out_shape=jax.ShapeDtypeStruct((B, N, O), jnp.float32),
)


def kernel(x, W, gamma, beta):
    idxT, y1t, y2t = _knn_call(x, x, W)
    idx2d = idxT[:, :K, :].reshape(TOT // CH, CH)
    table = y1t.reshape(B * N, 2 * O)
    g = _make_sc_gather()(table, idx2d)
    g4 = g.reshape(B, K, N, 2 * O)
    m, stats = _pool_call(g4, y2t)
    aux = (jnp.zeros((8, 128), jnp.float32)
           .at[0, :O].set(gamma).at[1, :O].set(beta))
    o = _norm_call(m, stats, aux)
    return o.transpose(0, 2, 1)


# speedup vs baseline: 1.0749x; 1.0749x over previous
import functools

import jax
import jax.numpy as jnp
from jax import lax
from jax.experimental import pallas as pl
from jax.experimental.pallas import tpu as pltpu
from jax.experimental.pallas import tpu_sc as plsc

B, C, N, K, O = 4, 64, 4096, 20, 64
KPAD = 32
T = 256
T2 = 128
T3 = 512
NW = 32
TOT = B * K * N
ROWS_PER_W = TOT // NW
CH = 128
NCH = ROWS_PER_W // CH


def _knn_body(xf_ref, xt_ref, w_ref, idx_ref, y1_ref, y2_ref, s_ref):
    b = pl.program_id(0)
    xb = xf_ref[0]
    xs = xt_ref[0]
    w = w_ref[...]
    inner = lax.dot_general(xs, xb, (((0,), (0,)), ((), ())),
                            preferred_element_type=jnp.float32)
    nf = jnp.sum(xb * xb, axis=0, keepdims=True)
    nt = jnp.sum(xs * xs, axis=0)[:, None]
    s_ref[...] = (2.0 * inner - nt) - nf

    w1pad = jnp.concatenate([w[:, :C], jnp.zeros((O, C), jnp.float32)], axis=0)
    y1_ref[0] = lax.dot_general(xs, w1pad, (((0,), (1,)), ((), ())),
                                preferred_element_type=jnp.float32)
    y2_ref[0] = lax.dot_general(xs, w[:, C:], (((0,), (1,)), ((), ())),
                                preferred_element_type=jnp.float32)

    col = lax.broadcasted_iota(jnp.int32, (T, N), 1)
    base = b * N

    def body(k, _):
        s = s_ref[...]
        mx = jnp.max(s, axis=1)
        eq = s == mx[:, None]
        a = jnp.min(jnp.where(eq, col, N), axis=1).astype(jnp.int32)
        idx_ref[0, pl.ds(k, 1), :] = (a + base)[None, :]
        s_ref[...] = jnp.where(col == a[:, None], -jnp.inf, s)
        return 0

    lax.fori_loop(0, K, body, 0)


_knn_call = pl.pallas_call(
    _knn_body,
    grid=(B, N // T),
    in_specs=[
        pl.BlockSpec((1, C, N), lambda b, j: (b, 0, 0)),
        pl.BlockSpec((1, C, T), lambda b, j: (b, 0, j)),
        pl.BlockSpec((O, 2 * C), lambda b, j: (0, 0)),
    ],
    out_specs=[
        pl.BlockSpec((1, KPAD, T), lambda b, j: (b, 0, j)),
        pl.BlockSpec((1, T, 2 * O), lambda b, j: (b, j, 0)),
        pl.BlockSpec((1, T, O), lambda b, j: (b, j, 0)),
    ],
    out_shape=[
        jax.ShapeDtypeStruct((B, KPAD, N), jnp.int32),
        jax.ShapeDtypeStruct((B, N, 2 * O), jnp.float32),
        jax.ShapeDtypeStruct((B, N, O), jnp.float32),
    ],
    scratch_shapes=[pltpu.VMEM((T, N), jnp.float32)],
)


@functools.cache
def _make_sc_gather():
    return functools.partial(
        pl.kernel,
        mesh=plsc.VectorSubcoreMesh(core_axis_name="c", subcore_axis_name="s"),
        out_type=jax.ShapeDtypeStruct((TOT, 2 * O), jnp.float32),
        scratch_types=[
            pltpu.VMEM((NCH, CH), jnp.int32),
            pltpu.VMEM((CH, 2 * O), jnp.float32),
            pltpu.VMEM((CH, 2 * O), jnp.float32),
            pltpu.SemaphoreType.DMA,
            pltpu.SemaphoreType.DMA,
        ],
    )(_sc_gather_body)


def _sc_gather_body(table_hbm, idx_hbm, out_hbm, idx_v, buf0, buf1, sem0, sem1):
    wid = lax.axis_index("s") * 2 + lax.axis_index("c")
    pltpu.sync_copy(idx_hbm.at[pl.ds(wid * NCH, NCH)], idx_v)
    base = wid * ROWS_PER_W

    def start(i, buf, sem):
        return pltpu.async_copy(table_hbm.at[idx_v.at[i]], buf, sem)

    def drain(i, buf, cp):
        cp.wait()
        off = pl.multiple_of(base + i * CH, CH)
        pltpu.sync_copy(buf, out_hbm.at[pl.ds(off, CH)])

    def body(i, _):
        cp0 = start(2 * i, buf0, sem0)
        cp1 = start(2 * i + 1, buf1, sem1)
        drain(2 * i, buf0, cp0)
        drain(2 * i + 1, buf1, cp1)
        return 0

    lax.fori_loop(0, NCH // 2, body, 0)


def _pool_body(g_ref, y2_ref, m_ref, st_ref):
    y2 = y2_ref[0]
    m = jnp.full((T2, O), -jnp.inf, jnp.float32)
    zs = jnp.zeros((T2, O), jnp.float32)
    zq = jnp.zeros((T2, O), jnp.float32)
    for k in range(K):
        z = g_ref[0, k, :, :O] + y2
        m = jnp.maximum(m, z)
        zs = zs + z
        zq = zq + z * z
    m_ref[0] = m
    s = jnp.sum(zs, axis=0)
    q = jnp.sum(zq, axis=0)
    zero = jnp.zeros((O,), jnp.float32)
    blk = jnp.concatenate([
        jnp.concatenate([s, zero])[None, :],
        jnp.concatenate([q, zero])[None, :],
        jnp.zeros((6, 128), jnp.float32),
    ], axis=0)
    first = (pl.program_id(0) == 0) & (pl.program_id(1) == 0)

    @pl.when(first)
    def _():
        st_ref[...] = jnp.zeros((8, 128), jnp.float32)

    st_ref[...] += blk


_pool_call = pl.pallas_call(
    _pool_body,
    grid=(B, N // T2),
    in_specs=[
        pl.BlockSpec((1, K, T2, 2 * O), lambda b, j: (b, 0, j, 0)),
        pl.BlockSpec((1, T2, O), lambda b, j: (b, j, 0)),
    ],
    out_specs=[
        pl.BlockSpec((1, T2, O), lambda b, j: (b, j, 0)),
        pl.BlockSpec((8, 128), lambda b, j: (0, 0)),
    ],
    out_shape=[
        jax.ShapeDtypeStruct((B, N, O), jnp.float32),
        jax.ShapeDtypeStruct((8, 128), jnp.float32),
    ],
)


def _norm_body(m_ref, st_ref, aux_ref, o_ref):
    m = m_ref[0]
    s = st_ref[0, :O]
    q = st_ref[1, :O]
    gamma = aux_ref[0, :O]
    beta = aux_ref[1, :O]
    cnt = jnp.float32(B * N * K)
    mean = s / cnt
    var = q / cnt - mean * mean
    inv = lax.rsqrt(var + 1e-5)
    scale = inv * gamma
    shift = beta - mean * scale
    o = m * scale[None, :] + shift[None, :]
    o_ref[0] = jnp.where(o > 0, o, 0.2 * o)


_norm_call = pl.pallas_call(
    _norm_body,
    grid=(B, N // T3),
    in_specs=[
        pl.BlockSpec((1, T3, O), lambda b, j: (b, j, 0)),
        pl.BlockSpec((8, 128), lambda b, j: (0, 0)),
        pl.BlockSpec((8, 128), lambda b, j: (0, 0)),
    ],
    out_specs=pl.BlockSpec((1, T3, O), lambda b, j: (b, j, 0)),
    out_shape=jax.ShapeDtypeStruct((B, N, O), jnp.float32),
)


def kernel(x, W, gamma, beta):
    idxT, y1t, y2t = _knn_call(x, x, W)
    idx2d = idxT[:, :K, :].reshape(TOT // CH, CH)
    table = y1t.reshape(B * N, 2 * O)
    g = _make_sc_gather()(table, idx2d)
    g4 = g.reshape(B, K, N, 2 * O)
    m, stats = _pool_call(g4, y2t)
    aux = (jnp.zeros((8, 128), jnp.float32)
           .at[0, :O].set(gamma).at[1, :O].set(beta))
    o = _norm_call(m, stats, aux)
    return o.transpose(0, 2, 1)


# speedup vs baseline: 1.1263x; 1.0479x over previous
import functools

import jax
import jax.numpy as jnp
from jax import lax
from jax.experimental import pallas as pl
from jax.experimental.pallas import tpu as pltpu
from jax.experimental.pallas import tpu_sc as plsc

B, C, N, K, O = 4, 64, 4096, 20, 64
KPAD = 32
T = 256
T2 = 128
T3 = 512
NW = 32
TOTB = K * N
ROWS_PER_W = TOTB // NW
CH = 128
NCH = ROWS_PER_W // CH


def _knn_body(xf_ref, xt_ref, w_ref, idx_ref, y1_ref, y2_ref, s_ref):
    xb = xf_ref[...]
    xs = xt_ref[...]
    w = w_ref[...]
    inner = lax.dot_general(xs, xb, (((0,), (0,)), ((), ())),
                            preferred_element_type=jnp.float32)
    nf = jnp.sum(xb * xb, axis=0, keepdims=True)
    nt = jnp.sum(xs * xs, axis=0)[:, None]
    s_ref[...] = (2.0 * inner - nt) - nf

    w1pad = jnp.concatenate([w[:, :C], jnp.zeros((O, C), jnp.float32)],
                            axis=0)
    y1_ref[...] = lax.dot_general(xs, w1pad, (((0,), (1,)), ((), ())),
                                  preferred_element_type=jnp.float32)
    y2_ref[...] = lax.dot_general(xs, w[:, C:], (((0,), (1,)), ((), ())),
                                  preferred_element_type=jnp.float32)

    col = lax.broadcasted_iota(jnp.int32, (T, N), 1)

    def body(k, _):
        s = s_ref[...]
        mx = jnp.max(s, axis=1)
        eq = s == mx[:, None]
        a = jnp.min(jnp.where(eq, col, N), axis=1).astype(jnp.int32)
        idx_ref[pl.ds(k, 1), :] = a[None, :]
        s_ref[...] = jnp.where(col == a[:, None], -jnp.inf, s)
        return 0

    lax.fori_loop(0, K, body, 0)


_knn_call = pl.pallas_call(
    _knn_body,
    grid=(N // T,),
    in_specs=[
        pl.BlockSpec((C, N), lambda j: (0, 0)),
        pl.BlockSpec((C, T), lambda j: (0, j)),
        pl.BlockSpec((O, 2 * C), lambda j: (0, 0)),
    ],
    out_specs=[
        pl.BlockSpec((KPAD, T), lambda j: (0, j)),
        pl.BlockSpec((T, 2 * O), lambda j: (j, 0)),
        pl.BlockSpec((T, O), lambda j: (j, 0)),
    ],
    out_shape=[
        jax.ShapeDtypeStruct((KPAD, N), jnp.int32),
        jax.ShapeDtypeStruct((N, 2 * O), jnp.float32),
        jax.ShapeDtypeStruct((N, O), jnp.float32),
    ],
    scratch_shapes=[pltpu.VMEM((T, N), jnp.float32)],
)


@functools.cache
def _make_sc_gather():
    return functools.partial(
        pl.kernel,
        mesh=plsc.VectorSubcoreMesh(core_axis_name="c", subcore_axis_name="s"),
        out_type=jax.ShapeDtypeStruct((TOTB, 2 * O), jnp.float32),
        scratch_types=[
            pltpu.VMEM((NCH, CH), jnp.int32),
            pltpu.VMEM((CH, 2 * O), jnp.float32),
            pltpu.VMEM((CH, 2 * O), jnp.float32),
            pltpu.SemaphoreType.DMA,
            pltpu.SemaphoreType.DMA,
        ],
    )(_sc_gather_body)


def _sc_gather_body(table_hbm, idx_hbm, out_hbm, idx_v, buf0, buf1, sem0, sem1):
    wid = lax.axis_index("s") * 2 + lax.axis_index("c")
    pltpu.sync_copy(idx_hbm.at[wid], idx_v)
    base = wid * ROWS_PER_W

    def start(i, buf, sem):
        return pltpu.async_copy(table_hbm.at[idx_v.at[i]], buf, sem)

    def drain(i, buf, cp):
        cp.wait()
        off = pl.multiple_of(base + i * CH, CH)
        pltpu.sync_copy(buf, out_hbm.at[pl.ds(off, CH)])

    def body(i, _):
        cp0 = start(2 * i, buf0, sem0)
        cp1 = start(2 * i + 1, buf1, sem1)
        drain(2 * i, buf0, cp0)
        drain(2 * i + 1, buf1, cp1)
        return 0

    lax.fori_loop(0, NCH // 2, body, 0)


def _pool_body(g_ref, y2_ref, m_ref, st_ref):
    y2 = y2_ref[...]
    m = jnp.full((T2, O), -jnp.inf, jnp.float32)
    zs = jnp.zeros((T2, O), jnp.float32)
    zq = jnp.zeros((T2, O), jnp.float32)
    for k in range(K):
        z = g_ref[k, :, :O] + y2
        m = jnp.maximum(m, z)
        zs = zs + z
        zq = zq + z * z
    m_ref[...] = m
    s = jnp.sum(zs, axis=0)
    q = jnp.sum(zq, axis=0)
    zero = jnp.zeros((O,), jnp.float32)
    blk = jnp.concatenate([
        jnp.concatenate([s, zero])[None, :],
        jnp.concatenate([q, zero])[None, :],
        jnp.zeros((6, 128), jnp.float32),
    ], axis=0)

    @pl.when(pl.program_id(0) == 0)
    def _():
        st_ref[...] = jnp.zeros((8, 128), jnp.float32)

    st_ref[...] += blk


_pool_call = pl.pallas_call(
    _pool_body,
    grid=(N // T2,),
    in_specs=[
        pl.BlockSpec((K, T2, 2 * O), lambda j: (0, j, 0)),
        pl.BlockSpec((T2, O), lambda j: (j, 0)),
    ],
    out_specs=[
        pl.BlockSpec((T2, O), lambda j: (j, 0)),
        pl.BlockSpec((8, 128), lambda j: (0, 0)),
    ],
    out_shape=[
        jax.ShapeDtypeStruct((N, O), jnp.float32),
        jax.ShapeDtypeStruct((8, 128), jnp.float32),
    ],
)


def _norm_body(m_ref, st_ref, aux_ref, o_ref):
    m = m_ref[...]
    s = st_ref[0, :O]
    q = st_ref[1, :O]
    gamma = aux_ref[0, :O]
    beta = aux_ref[1, :O]
    cnt = jnp.float32(B * N * K)
    mean = s / cnt
    var = q / cnt - mean * mean
    inv = lax.rsqrt(var + 1e-5)
    scale = inv * gamma
    shift = beta - mean * scale
    o = m * scale[None, :] + shift[None, :]
    o_ref[...] = jnp.where(o > 0, o, 0.2 * o)


_norm_call = pl.pallas_call(
    _norm_body,
    grid=(N // T3,),
    in_specs=[
        pl.BlockSpec((T3, O), lambda j: (j, 0)),
        pl.BlockSpec((8, 128), lambda j: (0, 0)),
        pl.BlockSpec((8, 128), lambda j: (0, 0)),
    ],
    out_specs=pl.BlockSpec((T3, O), lambda j: (j, 0)),
    out_shape=jax.ShapeDtypeStruct((N, O), jnp.float32),
)


def kernel(x, W, gamma, beta):
    scg = _make_sc_gather()
    ms, sts = [], []
    for b in range(B):
        xb = x[b]
        idxT, y1t, y2t = _knn_call(xb, xb, W)
        idx3d = idxT[:K, :].reshape(NW, NCH, CH)
        g = scg(y1t, idx3d)
        m_b, st_b = _pool_call(g.reshape(K, N, 2 * O), y2t)
        ms.append(m_b)
        sts.append(st_b)
    stats = sts[0] + sts[1] + sts[2] + sts[3]
    aux = (jnp.zeros((8, 128), jnp.float32)
           .at[0, :O].set(gamma).at[1, :O].set(beta))
    o = jnp.stack([_norm_call(m_b, stats, aux) for m_b in ms])
    return o.transpose(0, 2, 1)


# speedup vs baseline: 1.2001x; 1.0655x over previous
import functools

import jax
import jax.numpy as jnp
from jax import lax
from jax.experimental import pallas as pl
from jax.experimental.pallas import tpu as pltpu
from jax.experimental.pallas import tpu_sc as plsc

B, C, N, K, O = 4, 64, 4096, 20, 64
KPAD = 32
T = 512
T2 = 128
T3 = 512
NW = 32
TOTB = K * N
ROWS_PER_W = TOTB // NW
CH = 128
NCH = ROWS_PER_W // CH


def _knn_body(xf_ref, xt_ref, w_ref, idx_ref, y1_ref, y2_ref, s_ref):
    xb = xf_ref[...]
    xs = xt_ref[...]
    w = w_ref[...]
    inner = lax.dot_general(xs, xb, (((0,), (0,)), ((), ())),
                            preferred_element_type=jnp.float32)
    nf = jnp.sum(xb * xb, axis=0, keepdims=True)
    nt = jnp.sum(xs * xs, axis=0)[:, None]
    s_ref[...] = (2.0 * inner - nt) - nf

    w1pad = jnp.concatenate([w[:, :C], jnp.zeros((O, C), jnp.float32)],
                            axis=0)
    y1_ref[...] = lax.dot_general(xs, w1pad, (((0,), (1,)), ((), ())),
                                  preferred_element_type=jnp.float32)
    y2_ref[...] = lax.dot_general(xs, w[:, C:], (((0,), (1,)), ((), ())),
                                  preferred_element_type=jnp.float32)

    col = lax.broadcasted_iota(jnp.int32, (T, N), 1)

    def body(k, _):
        s = s_ref[...]
        mx = jnp.max(s, axis=1)
        eq = s == mx[:, None]
        a = jnp.min(jnp.where(eq, col, N), axis=1).astype(jnp.int32)
        idx_ref[pl.ds(k, 1), :] = a[None, :]
        s_ref[...] = jnp.where(col == a[:, None], -jnp.inf, s)
        return 0

    lax.fori_loop(0, K, body, 0)


_knn_call = pl.pallas_call(
    _knn_body,
    grid=(N // T,),
    in_specs=[
        pl.BlockSpec((C, N), lambda j: (0, 0)),
        pl.BlockSpec((C, T), lambda j: (0, j)),
        pl.BlockSpec((O, 2 * C), lambda j: (0, 0)),
    ],
    out_specs=[
        pl.BlockSpec((KPAD, T), lambda j: (0, j)),
        pl.BlockSpec((T, 2 * O), lambda j: (j, 0)),
        pl.BlockSpec((T, O), lambda j: (j, 0)),
    ],
    out_shape=[
        jax.ShapeDtypeStruct((KPAD, N), jnp.int32),
        jax.ShapeDtypeStruct((N, 2 * O), jnp.float32),
        jax.ShapeDtypeStruct((N, O), jnp.float32),
    ],
    scratch_shapes=[pltpu.VMEM((T, N), jnp.float32)],
)


@functools.cache
def _make_sc_gather():
    return functools.partial(
        pl.kernel,
        mesh=plsc.VectorSubcoreMesh(core_axis_name="c", subcore_axis_name="s"),
        out_type=jax.ShapeDtypeStruct((TOTB, 2 * O), jnp.float32),
        scratch_types=[
            pltpu.VMEM((NCH, CH), jnp.int32),
            pltpu.VMEM((CH, 2 * O), jnp.float32),
            pltpu.VMEM((CH, 2 * O), jnp.float32),
            pltpu.SemaphoreType.DMA,
            pltpu.SemaphoreType.DMA,
        ],
    )(_sc_gather_body)


def _sc_gather_body(table_hbm, idx_hbm, out_hbm, idx_v, buf0, buf1, sem0, sem1):
    wid = lax.axis_index("s") * 2 + lax.axis_index("c")
    pltpu.sync_copy(idx_hbm.at[wid], idx_v)
    base = wid * ROWS_PER_W

    def start(i, buf, sem):
        return pltpu.async_copy(table_hbm.at[idx_v.at[i]], buf, sem)

    def drain(i, buf, cp):
        cp.wait()
        off = pl.multiple_of(base + i * CH, CH)
        pltpu.sync_copy(buf, out_hbm.at[pl.ds(off, CH)])

    def body(i, _):
        cp0 = start(2 * i, buf0, sem0)
        cp1 = start(2 * i + 1, buf1, sem1)
        drain(2 * i, buf0, cp0)
        drain(2 * i + 1, buf1, cp1)
        return 0

    lax.fori_loop(0, NCH // 2, body, 0)


def _pool_body(g_ref, y2_ref, m_ref, st_ref):
    y2 = y2_ref[...]
    m = jnp.full((T2, O), -jnp.inf, jnp.float32)
    zs = jnp.zeros((T2, O), jnp.float32)
    zq = jnp.zeros((T2, O), jnp.float32)
    for k in range(K):
        z = g_ref[k, :, :O] + y2
        m = jnp.maximum(m, z)
        zs = zs + z
        zq = zq + z * z
    m_ref[...] = m
    s = jnp.sum(zs, axis=0)
    q = jnp.sum(zq, axis=0)
    zero = jnp.zeros((O,), jnp.float32)
    blk = jnp.concatenate([
        jnp.concatenate([s, zero])[None, :],
        jnp.concatenate([q, zero])[None, :],
        jnp.zeros((6, 128), jnp.float32),
    ], axis=0)

    @pl.when(pl.program_id(0) == 0)
    def _():
        st_ref[...] = jnp.zeros((8, 128), jnp.float32)

    st_ref[...] += blk


_pool_call = pl.pallas_call(
    _pool_body,
    grid=(N // T2,),
    in_specs=[
        pl.BlockSpec((K, T2, 2 * O), lambda j: (0, j, 0)),
        pl.BlockSpec((T2, O), lambda j: (j, 0)),
    ],
    out_specs=[
        pl.BlockSpec((T2, O), lambda j: (j, 0)),
        pl.BlockSpec((8, 128), lambda j: (0, 0)),
    ],
    out_shape=[
        jax.ShapeDtypeStruct((N, O), jnp.float32),
        jax.ShapeDtypeStruct((8, 128), jnp.float32),
    ],
)


def _norm_body(m_ref, st_ref, aux_ref, o_ref):
    m = m_ref[...]
    s = st_ref[0, :O]
    q = st_ref[1, :O]
    gamma = aux_ref[0, :O]
    beta = aux_ref[1, :O]
    cnt = jnp.float32(B * N * K)
    mean = s / cnt
    var = q / cnt - mean * mean
    inv = lax.rsqrt(var + 1e-5)
    scale = inv * gamma
    shift = beta - mean * scale
    o = m * scale[None, :] + shift[None, :]
    o_ref[...] = jnp.where(o > 0, o, 0.2 * o)


_norm_call = pl.pallas_call(
    _norm_body,
    grid=(N // T3,),
    in_specs=[
        pl.BlockSpec((T3, O), lambda j: (j, 0)),
        pl.BlockSpec((8, 128), lambda j: (0, 0)),
        pl.BlockSpec((8, 128), lambda j: (0, 0)),
    ],
    out_specs=pl.BlockSpec((T3, O), lambda j: (j, 0)),
    out_shape=jax.ShapeDtypeStruct((N, O), jnp.float32),
)


def kernel(x, W, gamma, beta):
    scg = _make_sc_gather()
    ms, sts = [], []
    for b in range(B):
        xb = x[b]
        idxT, y1t, y2t = _knn_call(xb, xb, W)
        idx3d = idxT[:K, :].reshape(NW, NCH, CH)
        g = scg(y1t, idx3d)
        m_b, st_b = _pool_call(g.reshape(K, N, 2 * O), y2t)
        ms.append(m_b)
        sts.append(st_b)
    stats = sts[0] + sts[1] + sts[2] + sts[3]
    aux = (jnp.zeros((8, 128), jnp.float32)
           .at[0, :O].set(gamma).at[1, :O].set(beta))
    o = jnp.stack([_norm_call(m_b, stats, aux) for m_b in ms])
    return o.transpose(0, 2, 1)


# speedup vs baseline: 1.2481x; 1.0400x over previous
import functools

import jax
import jax.numpy as jnp
from jax import lax
from jax.experimental import pallas as pl
from jax.experimental.pallas import tpu as pltpu
from jax.experimental.pallas import tpu_sc as plsc

B, C, N, K, O = 4, 64, 4096, 20, 64
KPAD = 32
T = 1024
T2 = 128
T3 = 512
NW = 32
TOTB = K * N
ROWS_PER_W = TOTB // NW
CH = 128
NCH = ROWS_PER_W // CH


def _knn_body(xf_ref, xt_ref, w_ref, idx_ref, y1_ref, y2_ref, s_ref):
    xb = xf_ref[...]
    xs = xt_ref[...]
    w = w_ref[...]
    inner = lax.dot_general(xs, xb, (((0,), (0,)), ((), ())),
                            preferred_element_type=jnp.float32)
    nf = jnp.sum(xb * xb, axis=0, keepdims=True)
    nt = jnp.sum(xs * xs, axis=0)[:, None]
    s_ref[...] = (2.0 * inner - nt) - nf

    w1pad = jnp.concatenate([w[:, :C], jnp.zeros((O, C), jnp.float32)],
                            axis=0)
    y1_ref[...] = lax.dot_general(xs, w1pad, (((0,), (1,)), ((), ())),
                                  preferred_element_type=jnp.float32)
    y2_ref[...] = lax.dot_general(xs, w[:, C:], (((0,), (1,)), ((), ())),
                                  preferred_element_type=jnp.float32)

    col = lax.broadcasted_iota(jnp.int32, (T, N), 1)

    def body(k, _):
        s = s_ref[...]
        mx = jnp.max(s, axis=1)
        eq = s == mx[:, None]
        a = jnp.min(jnp.where(eq, col, N), axis=1).astype(jnp.int32)
        idx_ref[pl.ds(k, 1), :] = a[None, :]
        s_ref[...] = jnp.where(col == a[:, None], -jnp.inf, s)
        return 0

    lax.fori_loop(0, K, body, 0)


_knn_call = pl.pallas_call(
    _knn_body,
    grid=(N // T,),
    in_specs=[
        pl.BlockSpec((C, N), lambda j: (0, 0)),
        pl.BlockSpec((C, T), lambda j: (0, j)),
        pl.BlockSpec((O, 2 * C), lambda j: (0, 0)),
    ],
    out_specs=[
        pl.BlockSpec((KPAD, T), lambda j: (0, j)),
        pl.BlockSpec((T, 2 * O), lambda j: (j, 0)),
        pl.BlockSpec((T, O), lambda j: (j, 0)),
    ],
    out_shape=[
        jax.ShapeDtypeStruct((KPAD, N), jnp.int32),
        jax.ShapeDtypeStruct((N, 2 * O), jnp.float32),
        jax.ShapeDtypeStruct((N, O), jnp.float32),
    ],
    scratch_shapes=[pltpu.VMEM((T, N), jnp.float32)],
)


@functools.cache
def _make_sc_gather():
    return functools.partial(
        pl.kernel,
        mesh=plsc.VectorSubcoreMesh(core_axis_name="c", subcore_axis_name="s"),
        out_type=jax.ShapeDtypeStruct((TOTB, 2 * O), jnp.float32),
        scratch_types=[
            pltpu.VMEM((NCH, CH), jnp.int32),
            pltpu.VMEM((CH, 2 * O), jnp.float32),
            pltpu.VMEM((CH, 2 * O), jnp.float32),
            pltpu.SemaphoreType.DMA,
            pltpu.SemaphoreType.DMA,
        ],
    )(_sc_gather_body)


def _sc_gather_body(table_hbm, idx_hbm, out_hbm, idx_v, buf0, buf1, sem0, sem1):
    wid = lax.axis_index("s") * 2 + lax.axis_index("c")
    pltpu.sync_copy(idx_hbm.at[wid], idx_v)
    base = wid * ROWS_PER_W

    def start(i, buf, sem):
        return pltpu.async_copy(table_hbm.at[idx_v.at[i]], buf, sem)

    def drain(i, buf, cp):
        cp.wait()
        off = pl.multiple_of(base + i * CH, CH)
        pltpu.sync_copy(buf, out_hbm.at[pl.ds(off, CH)])

    def body(i, _):
        cp0 = start(2 * i, buf0, sem0)
        cp1 = start(2 * i + 1, buf1, sem1)
        drain(2 * i, buf0, cp0)
        drain(2 * i + 1, buf1, cp1)
        return 0

    lax.fori_loop(0, NCH // 2, body, 0)


def _pool_body(g_ref, y2_ref, m_ref, st_ref):
    y2 = y2_ref[...]
    m = jnp.full((T2, O), -jnp.inf, jnp.float32)
    zs = jnp.zeros((T2, O), jnp.float32)
    zq = jnp.zeros((T2, O), jnp.float32)
    for k in range(K):
        z = g_ref[k, :, :O] + y2
        m = jnp.maximum(m, z)
        zs = zs + z
        zq = zq + z * z
    m_ref[...] = m
    s = jnp.sum(zs, axis=0)
    q = jnp.sum(zq, axis=0)
    zero = jnp.zeros((O,), jnp.float32)
    blk = jnp.concatenate([
        jnp.concatenate([s, zero])[None, :],
        jnp.concatenate([q, zero])[None, :],
        jnp.zeros((6, 128), jnp.float32),
    ], axis=0)

    @pl.when(pl.program_id(0) == 0)
    def _():
        st_ref[...] = jnp.zeros((8, 128), jnp.float32)

    st_ref[...] += blk


_pool_call = pl.pallas_call(
    _pool_body,
    grid=(N // T2,),
    in_specs=[
        pl.BlockSpec((K, T2, 2 * O), lambda j: (0, j, 0)),
        pl.BlockSpec((T2, O), lambda j: (j, 0)),
    ],
    out_specs=[
        pl.BlockSpec((T2, O), lambda j: (j, 0)),
        pl.BlockSpec((8, 128), lambda j: (0, 0)),
    ],
    out_shape=[
        jax.ShapeDtypeStruct((N, O), jnp.float32),
        jax.ShapeDtypeStruct((8, 128), jnp.float32),
    ],
)


def _norm_body(m_ref, st_ref, aux_ref, o_ref):
    m = m_ref[...]
    s = st_ref[0, :O]
    q = st_ref[1, :O]
    gamma = aux_ref[0, :O]
    beta = aux_ref[1, :O]
    cnt = jnp.float32(B * N * K)
    mean = s / cnt
    var = q / cnt - mean * mean
    inv = lax.rsqrt(var + 1e-5)
    scale = inv * gamma
    shift = beta - mean * scale
    o = m * scale[None, :] + shift[None, :]
    o_ref[...] = jnp.where(o > 0, o, 0.2 * o)


_norm_call = pl.pallas_call(
    _norm_body,
    grid=(N // T3,),
    in_specs=[
        pl.BlockSpec((T3, O), lambda j: (j, 0)),
        pl.BlockSpec((8, 128), lambda j: (0, 0)),
        pl.BlockSpec((8, 128), lambda j: (0, 0)),
    ],
    out_specs=pl.BlockSpec((T3, O), lambda j: (j, 0)),
    out_shape=jax.ShapeDtypeStruct((N, O), jnp.float32),
)


def kernel(x, W, gamma, beta):
    scg = _make_sc_gather()
    ms, sts = [], []
    for b in range(B):
        xb = x[b]
        idxT, y1t, y2t = _knn_call(xb, xb, W)
        idx3d = idxT[:K, :].reshape(NW, NCH, CH)
        g = scg(y1t, idx3d)
        m_b, st_b = _pool_call(g.reshape(K, N, 2 * O), y2t)
        ms.append(m_b)
        sts.append(st_b)
    stats = sts[0] + sts[1] + sts[2] + sts[3]
    aux = (jnp.zeros((8, 128), jnp.float32)
           .at[0, :O].set(gamma).at[1, :O].set(beta))
    o = jnp.stack([_norm_call(m_b, stats, aux) for m_b in ms])
    return o.transpose(0, 2, 1)
